# baseline (speedup 1.0000x reference)
import functools

import jax
import jax.numpy as jnp
from jax import lax
from jax.experimental import pallas as pl
from jax.experimental.pallas import tpu as pltpu
from jax.experimental.pallas import tpu_sc as plsc

_NC, _NS, _L = 2, 16, 16
_B = 16384
_COLS = 3
_CATS = 4096
_RANK = 16
_HRANK = _RANK // 2
_NPAIR = _NC * _NS // 2
_BPP = _B // _NPAIR
_GROUPS = _BPP // _L

_mesh = plsc.VectorSubcoreMesh(
    core_axis_name="c", subcore_axis_name="s",
    num_cores=_NC, num_subcores=_NS)


@functools.partial(
    pl.kernel,
    out_type=jax.ShapeDtypeStruct((_B,), jnp.float32),
    mesh=_mesh,
    compiler_params=pltpu.CompilerParams(
        needs_layout_passes=False, use_tc_tiling_on_sc=False),
    scratch_types=[
        pltpu.VMEM((_COLS * _BPP,), jnp.int32),
        pltpu.VMEM((_COLS * _BPP,), jnp.int32),
        pltpu.VMEM((_HRANK, _COLS * _CATS), jnp.float32),
        pltpu.VMEM((_COLS * _CATS,), jnp.float32),
        pltpu.VMEM((_BPP,), jnp.float32),
        pltpu.VMEM((_BPP,), jnp.float32),
        pltpu.VMEM_SHARED((_NS * _BPP,), jnp.float32),
        pltpu.SemaphoreType.DMA,
    ],
)
def _index_kernel(xo_hbm, yo_hbm, sft_hbm, stds_hbm, out_hbm,
                  xi_v, yi_v, tab_v, stds_v, pacc_v, tmp_v, shared, sem):
    c = lax.axis_index("c")
    s = lax.axis_index("s")
    h = s & 1
    pair = c * (_NS // 2) + (s >> 1)
    base = pair * _BPP

    _Q = _HRANK // 2
    tab_copies = [
        pltpu.async_copy(
            sft_hbm.at[pl.ds(h * _HRANK + q * _Q, _Q), :],
            tab_v.at[pl.ds(q * _Q, _Q), :], sem)
        for q in range(2)
    ]
    pltpu.sync_copy(stds_hbm, stds_v)
    for i in range(_COLS):
        pltpu.sync_copy(xo_hbm.at[pl.ds(i * _B + base, _BPP)],
                        xi_v.at[pl.ds(i * _BPP, _BPP)])
        pltpu.sync_copy(yo_hbm.at[pl.ds(i * _B + base, _BPP)],
                        yi_v.at[pl.ds(i * _BPP, _BPP)])

    iota = lax.iota(jnp.int32, _L)
    zero = jnp.zeros((_L,), jnp.float32)

    @pl.when(h == 0)
    def _diag():
        def diag_body(g, carry):
            b0 = g * _L
            acc = zero
            for i in range(_COLS):
                xv = xi_v[pl.ds(i * _BPP + b0, _L)]
                yv = yi_v[pl.ds(i * _BPP + b0, _L)]
                sv = plsc.load_gather(stds_v, [xv])
                acc = acc + jnp.where(xv == yv, sv * sv, zero)
            pacc_v[pl.ds(b0, _L)] = acc
            return carry
        lax.fori_loop(0, _GROUPS, diag_body, 0, unroll=2)

    @pl.when(h != 0)
    def _zero():
        def zero_body(g, carry):
            pacc_v[pl.ds(g * _L, _L)] = zero
            return carry
        lax.fori_loop(0, _GROUPS, zero_body, 0, unroll=2)

    for q in range(2):
        tab_copies[q].wait()

        def body(g, carry, q=q):
            b0 = g * _L
            accs = [pacc_v[pl.ds(b0, _L)], zero, zero, zero]
            for i in range(_COLS):
                xv = xi_v[pl.ds(i * _BPP + b0, _L)]
                yv = yi_v[pl.ds(i * _BPP + b0, _L)]
                for r in range(q * _Q, (q + 1) * _Q):
                    rr = jnp.full((_L,), r, jnp.int32)
                    fx = plsc.load_gather(tab_v, [rr, xv])
                    fy = plsc.load_gather(tab_v, [rr, yv])
                    p = fx * fy
                    accs[r % 4] = accs[r % 4] + p * p
            pacc_v[pl.ds(b0, _L)] = (accs[0] + accs[1]) + (accs[2] + accs[3])
            return carry

        lax.fori_loop(0, _GROUPS, body, 0, unroll=2)

    pltpu.sync_copy(pacc_v, shared.at[pl.ds(s * _BPP, _BPP)])
    plsc.subcore_barrier()

    @pl.when(h == 0)
    def _combine():
        pltpu.sync_copy(shared.at[pl.ds((s + 1) * _BPP, _BPP)], tmp_v)

        def add_body(g, carry):
            b0 = g * _L
            pacc_v[pl.ds(b0, _L)] = (
                pacc_v[pl.ds(b0, _L)] + tmp_v[pl.ds(b0, _L)])
            return carry
        lax.fori_loop(0, _GROUPS, add_body, 0, unroll=4)
        pltpu.sync_copy(pacc_v, out_hbm.at[pl.ds(base, _BPP)])


def kernel(x, y, sqrt_covar_factors, stds):
    off = jnp.arange(_COLS, dtype=jnp.int32) * _CATS
    xo = (x + off[None, :]).T.reshape(_COLS * _B)
    yo = (y + off[None, :]).T.reshape(_COLS * _B)
    sft = jnp.transpose(sqrt_covar_factors, (2, 0, 1)).reshape(
        _RANK, _COLS * _CATS)
    stds_flat = stds.reshape(_COLS * _CATS)
    return _index_kernel(xo, yo, sft, stds_flat)


# speedup vs baseline: 1.0202x; 1.0202x over previous
import functools

import jax
import jax.numpy as jnp
from jax import lax
from jax.experimental import pallas as pl
from jax.experimental.pallas import tpu as pltpu
from jax.experimental.pallas import tpu_sc as plsc

_NC, _NS, _L = 2, 16, 16
_B = 16384
_COLS = 3
_CATS = 4096
_RANK = 16
_HRANK = _RANK // 2
_NPAIR = _NC * _NS // 2
_BPP = _B // _NPAIR
_GROUPS = _BPP // _L

_mesh = plsc.VectorSubcoreMesh(
    core_axis_name="c", subcore_axis_name="s",
    num_cores=_NC, num_subcores=_NS)


@functools.partial(
    pl.kernel,
    out_type=jax.ShapeDtypeStruct((_B,), jnp.float32),
    mesh=_mesh,
    compiler_params=pltpu.CompilerParams(
        needs_layout_passes=False, use_tc_tiling_on_sc=False),
    scratch_types=[
        pltpu.VMEM((_COLS * _BPP,), jnp.int32),
        pltpu.VMEM((_COLS * _BPP,), jnp.int32),
        pltpu.VMEM((_HRANK, _COLS * _CATS), jnp.float32),
        pltpu.VMEM((_COLS * _CATS,), jnp.float32),
        pltpu.VMEM((_BPP,), jnp.float32),
        pltpu.VMEM((_BPP,), jnp.float32),
        pltpu.VMEM_SHARED((_NS * _BPP,), jnp.float32),
        pltpu.SemaphoreType.DMA,
    ],
)
def _index_kernel(xo_hbm, yo_hbm, sft_hbm, stds_hbm, out_hbm,
                  xi_v, yi_v, tab_v, stds_v, pacc_v, tmp_v, shared, sem):
    c = lax.axis_index("c")
    s = lax.axis_index("s")
    h = s & 1
    pair = c * (_NS // 2) + (s >> 1)
    base = pair * _BPP

    tab_copy = pltpu.async_copy(
        sft_hbm.at[pl.ds(h * _HRANK, _HRANK), :], tab_v, sem)
    pltpu.sync_copy(stds_hbm, stds_v)
    for i in range(_COLS):
        pltpu.sync_copy(xo_hbm.at[pl.ds(i * _B + base, _BPP)],
                        xi_v.at[pl.ds(i * _BPP, _BPP)])
        pltpu.sync_copy(yo_hbm.at[pl.ds(i * _B + base, _BPP)],
                        yi_v.at[pl.ds(i * _BPP, _BPP)])

    iota = lax.iota(jnp.int32, _L)
    zero = jnp.zeros((_L,), jnp.float32)

    @pl.when(h == 0)
    def _diag():
        def diag_body(g, carry):
            b0 = g * _L
            acc = zero
            for i in range(_COLS):
                xv = xi_v[pl.ds(i * _BPP + b0, _L)]
                yv = yi_v[pl.ds(i * _BPP + b0, _L)]
                sv = plsc.load_gather(stds_v, [xv])
                acc = acc + jnp.where(xv == yv, sv * sv, zero)
            pacc_v[pl.ds(b0, _L)] = acc
            return carry
        lax.fori_loop(0, _GROUPS, diag_body, 0, unroll=2)

    @pl.when(h != 0)
    def _zero():
        def zero_body(g, carry):
            pacc_v[pl.ds(g * _L, _L)] = zero
            return carry
        lax.fori_loop(0, _GROUPS, zero_body, 0, unroll=2)

    tab_copy.wait()

    def body(g, carry):
        b0 = g * _L
        accs = [pacc_v[pl.ds(b0, _L)], zero, zero, zero]
        for i in range(_COLS):
            xv = xi_v[pl.ds(i * _BPP + b0, _L)]
            yv = yi_v[pl.ds(i * _BPP + b0, _L)]
            for r in range(_HRANK):
                rr = jnp.full((_L,), r, jnp.int32)
                fx = plsc.load_gather(tab_v, [rr, xv])
                fy = plsc.load_gather(tab_v, [rr, yv])
                p = fx * fy
                accs[r % 4] = accs[r % 4] + p * p
        pacc_v[pl.ds(b0, _L)] = (accs[0] + accs[1]) + (accs[2] + accs[3])
        return carry

    lax.fori_loop(0, _GROUPS, body, 0, unroll=4)

    pltpu.sync_copy(pacc_v, shared.at[pl.ds(s * _BPP, _BPP)])
    plsc.subcore_barrier()

    @pl.when(h == 0)
    def _combine():
        pltpu.sync_copy(shared.at[pl.ds((s + 1) * _BPP, _BPP)], tmp_v)

        def add_body(g, carry):
            b0 = g * _L
            pacc_v[pl.ds(b0, _L)] = (
                pacc_v[pl.ds(b0, _L)] + tmp_v[pl.ds(b0, _L)])
            return carry
        lax.fori_loop(0, _GROUPS, add_body, 0, unroll=4)
        pltpu.sync_copy(pacc_v, out_hbm.at[pl.ds(base, _BPP)])


def kernel(x, y, sqrt_covar_factors, stds):
    off = jnp.arange(_COLS, dtype=jnp.int32) * _CATS
    xo = (x + off[None, :]).T.reshape(_COLS * _B)
    yo = (y + off[None, :]).T.reshape(_COLS * _B)
    sft = jnp.transpose(sqrt_covar_factors, (2, 0, 1)).reshape(
        _RANK, _COLS * _CATS)
    stds_flat = stds.reshape(_COLS * _CATS)
    return _index_kernel(xo, yo, sft, stds_flat)


# speedup vs baseline: 1.0324x; 1.0120x over previous
import functools

import jax
import jax.numpy as jnp
from jax import lax
from jax.experimental import pallas as pl
from jax.experimental.pallas import tpu as pltpu
from jax.experimental.pallas import tpu_sc as plsc

_NC, _NS, _L = 2, 16, 16
_B = 16384
_COLS = 3
_CATS = 4096
_RANK = 16
_HRANK = _RANK // 2
_NPAIR = _NC * _NS // 2
_BPP = _B // _NPAIR
_GROUPS = _BPP // _L

_mesh = plsc.VectorSubcoreMesh(
    core_axis_name="c", subcore_axis_name="s",
    num_cores=_NC, num_subcores=_NS)


@functools.partial(
    pl.kernel,
    out_type=jax.ShapeDtypeStruct((_B,), jnp.float32),
    mesh=_mesh,
    compiler_params=pltpu.CompilerParams(
        needs_layout_passes=False, use_tc_tiling_on_sc=False),
    scratch_types=[
        pltpu.VMEM((_COLS * _BPP,), jnp.int32),
        pltpu.VMEM((_COLS * _BPP,), jnp.int32),
        pltpu.VMEM((_HRANK, _COLS * _CATS), jnp.float32),
        pltpu.VMEM((_COLS * _CATS,), jnp.float32),
        pltpu.VMEM((_BPP,), jnp.float32),
        pltpu.VMEM((_BPP,), jnp.float32),
        pltpu.VMEM_SHARED((_NS * _BPP,), jnp.float32),
        pltpu.SemaphoreType.DMA,
    ],
)
def _index_kernel(xo_hbm, yo_hbm, sft_hbm, stds_hbm, out_hbm,
                  xi_v, yi_v, tab_v, stds_v, pacc_v, tmp_v, shared, sem):
    c = lax.axis_index("c")
    s = lax.axis_index("s")
    h = s & 1
    pair = c * (_NS // 2) + (s >> 1)
    base = pair * _BPP

    tab_copy = pltpu.async_copy(
        sft_hbm.at[pl.ds(h * _HRANK, _HRANK), :], tab_v, sem)
    pltpu.sync_copy(stds_hbm, stds_v)
    for i in range(_COLS):
        pltpu.sync_copy(xo_hbm.at[pl.ds(i * _B + base, _BPP)],
                        xi_v.at[pl.ds(i * _BPP, _BPP)])
        pltpu.sync_copy(yo_hbm.at[pl.ds(i * _B + base, _BPP)],
                        yi_v.at[pl.ds(i * _BPP, _BPP)])

    iota = lax.iota(jnp.int32, _L)
    zero = jnp.zeros((_L,), jnp.float32)

    @pl.when(h == 0)
    def _diag():
        def diag_body(g, carry):
            b0 = g * _L
            acc = zero
            for i in range(_COLS):
                xv = xi_v[pl.ds(i * _BPP + b0, _L)]
                yv = yi_v[pl.ds(i * _BPP + b0, _L)]
                sv = plsc.load_gather(stds_v, [xv])
                acc = acc + jnp.where(xv == yv, sv * sv, zero)
            pacc_v[pl.ds(b0, _L)] = acc
            return carry
        lax.fori_loop(0, _GROUPS, diag_body, 0, unroll=2)

    @pl.when(h != 0)
    def _zero():
        def zero_body(g, carry):
            pacc_v[pl.ds(g * _L, _L)] = zero
            return carry
        lax.fori_loop(0, _GROUPS, zero_body, 0, unroll=2)

    tab_copy.wait()

    def body(g, carry):
        b0 = g * _L
        accs = [pacc_v[pl.ds(b0, _L)], zero, zero, zero]
        for i in range(_COLS):
            xv = xi_v[pl.ds(i * _BPP + b0, _L)]
            yv = yi_v[pl.ds(i * _BPP + b0, _L)]
            for r in range(_HRANK):
                rr = jnp.full((_L,), r, jnp.int32)
                fx = plsc.load_gather(tab_v, [rr, xv])
                fy = plsc.load_gather(tab_v, [rr, yv])
                p = fx * fy
                accs[r % 4] = accs[r % 4] + p * p
        pacc_v[pl.ds(b0, _L)] = (accs[0] + accs[1]) + (accs[2] + accs[3])
        return carry

    lax.fori_loop(0, _GROUPS, body, 0, unroll=2)

    pltpu.sync_copy(pacc_v, shared.at[pl.ds(s * _BPP, _BPP)])
    plsc.subcore_barrier()

    @pl.when(h == 0)
    def _combine():
        pltpu.sync_copy(shared.at[pl.ds((s + 1) * _BPP, _BPP)], tmp_v)

        def add_body(g, carry):
            b0 = g * _L
            pacc_v[pl.ds(b0, _L)] = (
                pacc_v[pl.ds(b0, _L)] + tmp_v[pl.ds(b0, _L)])
            return carry
        lax.fori_loop(0, _GROUPS, add_body, 0, unroll=4)
        pltpu.sync_copy(pacc_v, out_hbm.at[pl.ds(base, _BPP)])


def kernel(x, y, sqrt_covar_factors, stds):
    off = jnp.arange(_COLS, dtype=jnp.int32) * _CATS
    xo = (x + off[None, :]).T.reshape(_COLS * _B)
    yo = (y + off[None, :]).T.reshape(_COLS * _B)
    sft = jnp.transpose(sqrt_covar_factors, (2, 0, 1)).reshape(
        _RANK, _COLS * _CATS)
    stds_flat = stds.reshape(_COLS * _CATS)
    return _index_kernel(xo, yo, sft, stds_flat)
